# Initial kernel scaffold; baseline (speedup 1.0000x reference)
import jax
import jax.numpy as jnp
from jax.experimental import pallas as pl


def kernel(tables, W1, b1, gamma, beta, indices):
    raise NotImplementedError("write your pallas kernel here")


# speedup vs baseline: 1.6933x; 1.6933x over previous
import functools

import jax
import jax.numpy as jnp
from jax import lax
from jax.experimental import pallas as pl
from jax.experimental.pallas import tpu as pltpu
from jax.experimental.pallas import tpu_sc as plsc

NC, NS, LANES = 2, 16, 16
NW = NC * NS


def _emb_sum(tables_flat, idx2, P, V, H, NF):
    CHUNK = P // NW
    S = 64
    NBLK = CHUNK // S

    mesh = plsc.VectorSubcoreMesh(core_axis_name="c", subcore_axis_name="s")

    @functools.partial(
        pl.kernel,
        out_type=jax.ShapeDtypeStruct((P, H), jnp.float32),
        mesh=mesh,
        scratch_types=(
            [pltpu.VMEM((CHUNK,), jnp.int32) for _ in range(NF)]
            + [pltpu.VMEM((S, H), jnp.float32) for _ in range(NF)]
            + [pltpu.SemaphoreType.DMA]
        ),
        compiler_params=pltpu.CompilerParams(use_tc_tiling_on_sc=False),
    )
    def emb_kernel(tab_hbm, idx_hbm, out_hbm, *scratch):
        idx_v = scratch[:NF]
        rows_v = scratch[NF:2 * NF]
        sem = scratch[2 * NF]
        wid = lax.axis_index("s") * NC + lax.axis_index("c")
        base0 = wid * CHUNK

        idescs = [
            pltpu.async_copy(idx_hbm.at[pl.ds(f * P + base0, CHUNK)], idx_v[f], sem)
            for f in range(NF)
        ]
        for d in idescs:
            d.wait()

        for f in range(1, NF):
            off = f * V

            @pl.loop(0, CHUNK // LANES)
            def _(j, f=f, off=off):
                sl = pl.ds(j * LANES, LANES)
                idx_v[f][sl] = idx_v[f][sl] + off

        @pl.loop(0, NBLK)
        def _(blk):
            bs = blk * S
            descs = [
                pltpu.async_copy(
                    tab_hbm.at[idx_v[f].at[pl.ds(bs, S)]], rows_v[f], sem
                )
                for f in range(NF)
            ]
            for d in descs:
                d.wait()

            @pl.loop(0, S)
            def _(p):
                for j in range(H // LANES):
                    sl = pl.ds(j * LANES, LANES)
                    v = rows_v[0][p, sl]
                    for f in range(1, NF):
                        v = v + rows_v[f][p, sl]
                    rows_v[0][p, sl] = v

            pltpu.sync_copy(rows_v[0], out_hbm.at[pl.ds(base0 + bs, S)])

    return emb_kernel(tables_flat, idx2)


def _mlp(emb, W1, b1, gamma, beta):
    P, H = emb.shape
    BR = 2048

    def body(x_ref, w_ref, b_ref, g_ref, bt_ref, o_ref):
        x = x_ref[...]
        h = jnp.dot(x, w_ref[...], preferred_element_type=jnp.float32) + b_ref[...]
        mu = jnp.mean(h, axis=-1, keepdims=True)
        var = jnp.mean(jnp.square(h - mu), axis=-1, keepdims=True)
        hn = (h - mu) * lax.rsqrt(var + 1e-5) * g_ref[...] + bt_ref[...]
        o_ref[...] = jnp.maximum(hn, 0.0)

    return pl.pallas_call(
        body,
        grid=(P // BR,),
        in_specs=[
            pl.BlockSpec((BR, H), lambda i: (i, 0)),
            pl.BlockSpec((H, H), lambda i: (0, 0)),
            pl.BlockSpec((1, H), lambda i: (0, 0)),
            pl.BlockSpec((1, H), lambda i: (0, 0)),
            pl.BlockSpec((1, H), lambda i: (0, 0)),
        ],
        out_specs=pl.BlockSpec((BR, H), lambda i: (i, 0)),
        out_shape=jax.ShapeDtypeStruct((P, H), jnp.float32),
    )(emb, W1, b1.reshape(1, H), gamma.reshape(1, H), beta.reshape(1, H))


def kernel(tables, W1, b1, gamma, beta, indices):
    NF, V, H = tables.shape
    _, Bb, Ll = indices.shape
    P = Bb * Ll
    emb = _emb_sum(
        tables.reshape(NF * V, H),
        indices.reshape(NF * P).astype(jnp.int32),
        P, V, H, NF,
    )
    out = _mlp(emb, W1, b1, gamma, beta)
    return out.reshape(Bb, Ll, H)


# speedup vs baseline: 1.7246x; 1.0185x over previous
import functools

import jax
import jax.numpy as jnp
from jax import lax
from jax.experimental import pallas as pl
from jax.experimental.pallas import tpu as pltpu
from jax.experimental.pallas import tpu_sc as plsc

NC, NS, LANES = 2, 16, 16
NW = NC * NS
OW = 128


def _emb_sum(tables_flat, idx2, P, V, H, NF):
    CHUNK = P // NW
    S = 80
    NBLK = CHUNK // S

    mesh = plsc.VectorSubcoreMesh(core_axis_name="c", subcore_axis_name="s")

    @functools.partial(
        pl.kernel,
        out_type=jax.ShapeDtypeStruct((P, OW), jnp.float32),
        mesh=mesh,
        scratch_types=(
            [pltpu.VMEM((CHUNK,), jnp.int32) for _ in range(NF)]
            + [pltpu.VMEM((S, H), jnp.float32) for _ in range(2 * NF)]
            + [pltpu.VMEM((S, OW), jnp.float32)]
            + [pltpu.SemaphoreType.DMA for _ in range(3)]
        ),
        compiler_params=pltpu.CompilerParams(use_tc_tiling_on_sc=False),
    )
    def emb_kernel(tab_hbm, idx_hbm, out_hbm, *scratch):
        idx_v = scratch[:NF]
        rows_v = [scratch[NF:2 * NF], scratch[2 * NF:3 * NF]]
        acc = scratch[3 * NF]
        isem, semA, semB = scratch[3 * NF + 1:]
        sems = [semA, semB]
        wid = lax.axis_index("s") * NC + lax.axis_index("c")
        base0 = wid * CHUNK

        idescs = [
            pltpu.async_copy(idx_hbm.at[pl.ds(f * P + base0, CHUNK)], idx_v[f], isem)
            for f in range(NF)
        ]
        for d in idescs:
            d.wait()

        @pl.loop(0, CHUNK // LANES)
        def _(j):
            sl = pl.ds(j * LANES, LANES)
            for f in range(1, NF):
                idx_v[f][sl] = idx_v[f][sl] + f * V

        def fire(blk, st):
            bs = blk * S
            return [
                pltpu.async_copy(
                    tab_hbm.at[idx_v[f].at[pl.ds(bs, S)]], rows_v[st][f], sems[st]
                )
                for f in range(NF)
            ]

        descs = fire(0, 0)
        for k in range(NBLK):
            st = k % 2
            nxt = None
            if k + 1 < NBLK:
                nxt = fire(k + 1, (k + 1) % 2)
            for d in descs:
                d.wait()

            rv = rows_v[st]

            @pl.loop(0, S)
            def _(p, rv=rv):
                for j in range(H // LANES):
                    sl = pl.ds(j * LANES, LANES)
                    v = rv[0][p, sl]
                    for f in range(1, NF):
                        v = v + rv[f][p, sl]
                    acc[p, sl] = v

            pltpu.sync_copy(acc, out_hbm.at[pl.ds(base0 + k * S, S)])
            descs = nxt

    return emb_kernel(tables_flat, idx2)


def _mlp(emb, W1, b1, gamma, beta, H):
    P = emb.shape[0]
    BR = 2048

    def body(x_ref, w_ref, b_ref, g_ref, bt_ref, o_ref):
        x = x_ref[:, 0:H]
        h = jnp.dot(x, w_ref[...], preferred_element_type=jnp.float32) + b_ref[...]
        mu = jnp.mean(h, axis=-1, keepdims=True)
        var = jnp.mean(jnp.square(h - mu), axis=-1, keepdims=True)
        hn = (h - mu) * lax.rsqrt(var + 1e-5) * g_ref[...] + bt_ref[...]
        o_ref[...] = jnp.maximum(hn, 0.0)

    return pl.pallas_call(
        body,
        grid=(P // BR,),
        in_specs=[
            pl.BlockSpec((BR, OW), lambda i: (i, 0)),
            pl.BlockSpec((H, H), lambda i: (0, 0)),
            pl.BlockSpec((1, H), lambda i: (0, 0)),
            pl.BlockSpec((1, H), lambda i: (0, 0)),
            pl.BlockSpec((1, H), lambda i: (0, 0)),
        ],
        out_specs=pl.BlockSpec((BR, H), lambda i: (i, 0)),
        out_shape=jax.ShapeDtypeStruct((P, H), jnp.float32),
    )(emb, W1, b1.reshape(1, H), gamma.reshape(1, H), beta.reshape(1, H))


def kernel(tables, W1, b1, gamma, beta, indices):
    NF, V, H = tables.shape
    _, Bb, Ll = indices.shape
    P = Bb * Ll
    emb = _emb_sum(
        tables.reshape(NF * V, H),
        indices.reshape(NF * P).astype(jnp.int32),
        P, V, H, NF,
    )
    out = _mlp(emb, W1, b1, gamma, beta, H)
    return out.reshape(Bb, Ll, H)


# speedup vs baseline: 1.7258x; 1.0007x over previous
import functools

import jax
import jax.numpy as jnp
from jax import lax
from jax.experimental import pallas as pl
from jax.experimental.pallas import tpu as pltpu
from jax.experimental.pallas import tpu_sc as plsc

NC, NS, LANES = 2, 16, 16
NW = NC * NS
OW = 128


def _emb_sum(tables, idx2, P, V, H, NF):
    CHUNK = P // NW
    S = 80
    NBLK = CHUNK // S

    mesh = plsc.VectorSubcoreMesh(core_axis_name="c", subcore_axis_name="s")

    @functools.partial(
        pl.kernel,
        out_type=jax.ShapeDtypeStruct((P, OW), jnp.float32),
        mesh=mesh,
        scratch_types=(
            [pltpu.VMEM((CHUNK,), jnp.int32) for _ in range(NF)]
            + [pltpu.VMEM((S, H), jnp.float32) for _ in range(2 * NF)]
            + [pltpu.VMEM((S, OW), jnp.float32)]
            + [pltpu.SemaphoreType.DMA for _ in range(3)]
        ),
        compiler_params=pltpu.CompilerParams(use_tc_tiling_on_sc=False),
    )
    def emb_kernel(tab_hbm, idx_hbm, out_hbm, *scratch):
        idx_v = scratch[:NF]
        rows_v = [scratch[NF:2 * NF], scratch[2 * NF:3 * NF]]
        acc = scratch[3 * NF]
        isem, semA, semB = scratch[3 * NF + 1:]
        sems = [semA, semB]
        wid = lax.axis_index("s") * NC + lax.axis_index("c")
        base0 = wid * CHUNK

        idescs = [
            pltpu.async_copy(idx_hbm.at[pl.ds(f * P + base0, CHUNK)], idx_v[f], isem)
            for f in range(NF)
        ]
        for d in idescs:
            d.wait()

        def fire(blk, st):
            bs = blk * S
            return [
                pltpu.async_copy(
                    tab_hbm.at[f].at[idx_v[f].at[pl.ds(bs, S)]],
                    rows_v[st][f],
                    sems[st],
                )
                for f in range(NF)
            ]

        descs = fire(0, 0)
        for k in range(NBLK):
            st = k % 2
            nxt = None
            if k + 1 < NBLK:
                nxt = fire(k + 1, (k + 1) % 2)
            for d in descs:
                d.wait()

            rv = rows_v[st]

            @pl.loop(0, S)
            def _(p, rv=rv):
                for j in range(H // LANES):
                    sl = pl.ds(j * LANES, LANES)
                    v = rv[0][p, sl]
                    for f in range(1, NF):
                        v = v + rv[f][p, sl]
                    acc[p, sl] = v

            pltpu.sync_copy(acc, out_hbm.at[pl.ds(base0 + k * S, S)])
            descs = nxt

    return emb_kernel(tables, idx2)


def _mlp(emb, W1, b1, gamma, beta, H):
    P = emb.shape[0]
    BR = 2048

    def body(x_ref, w_ref, b_ref, g_ref, bt_ref, o_ref):
        x = x_ref[:, 0:H]
        h = jnp.dot(x, w_ref[...], preferred_element_type=jnp.float32) + b_ref[...]
        mu = jnp.mean(h, axis=-1, keepdims=True)
        var = jnp.mean(jnp.square(h - mu), axis=-1, keepdims=True)
        hn = (h - mu) * lax.rsqrt(var + 1e-5) * g_ref[...] + bt_ref[...]
        o_ref[...] = jnp.maximum(hn, 0.0)

    return pl.pallas_call(
        body,
        grid=(P // BR,),
        in_specs=[
            pl.BlockSpec((BR, OW), lambda i: (i, 0)),
            pl.BlockSpec((H, H), lambda i: (0, 0)),
            pl.BlockSpec((1, H), lambda i: (0, 0)),
            pl.BlockSpec((1, H), lambda i: (0, 0)),
            pl.BlockSpec((1, H), lambda i: (0, 0)),
        ],
        out_specs=pl.BlockSpec((BR, H), lambda i: (i, 0)),
        out_shape=jax.ShapeDtypeStruct((P, H), jnp.float32),
    )(emb, W1, b1.reshape(1, H), gamma.reshape(1, H), beta.reshape(1, H))


def kernel(tables, W1, b1, gamma, beta, indices):
    NF, V, H = tables.shape
    _, Bb, Ll = indices.shape
    P = Bb * Ll
    emb = _emb_sum(
        tables,
        indices.reshape(NF * P).astype(jnp.int32),
        P, V, H, NF,
    )
    out = _mlp(emb, W1, b1, gamma, beta, H)
    return out.reshape(Bb, Ll, H)
